# baseline (speedup 1.0000x reference)
import numpy as np
import jax
import jax.numpy as jnp
from jax import lax
from jax.experimental import pallas as pl
from jax.experimental.pallas import tpu as pltpu
from jax.experimental.pallas import tpu_sc as plsc

D_MODEL = 256
N_LEVELS = 4
N_HEADS = 8
N_POINTS = 4
HEAD_DIM = 32
_SHAPES = np.array([[128, 128], [64, 64], [32, 32], [16, 16]], dtype=np.int64)
LEN = int(_SHAPES.prod(1).sum())
_STARTS = np.concatenate([np.zeros((1,), np.int64), _SHAPES.prod(1).cumsum()[:-1]])

QBLK = 256
NQB = LEN // QBLK

_c = np.arange(N_HEADS * N_LEVELS * N_POINTS)
_l_of_c = (_c // N_POINTS) % N_LEVELS
_h_of_c = _c // (N_LEVELS * N_POINTS)
_Wl = _SHAPES[_l_of_c, 1].astype(np.float32)
_Hl = _SHAPES[_l_of_c, 0].astype(np.float32)
_BASE = (_STARTS[_l_of_c] * N_HEADS + _h_of_c).astype(np.float32)
_ExW = np.zeros((8, 128), np.float32)
_EyH = np.zeros((8, 128), np.float32)
for _cc in range(128):
    _ExW[_l_of_c[_cc] * 2 + 0, _cc] = _Wl[_cc]
    _EyH[_l_of_c[_cc] * 2 + 1, _cc] = _Hl[_cc]
_SEG = (np.arange(128)[:, None] // 16 == np.arange(128)[None, :] // 16).astype(np.float32)
_j32 = np.arange(32)
_orig = np.where(_j32 < 16, 2 * _j32, 2 * (_j32 - 16) + 1)
_PERM = (np.arange(D_MODEL) // 32) * 32 + _orig[np.arange(D_MODEL) % 32]
_CV = np.zeros((8, 128), np.float32)
_CV[0] = _Wl
_CV[1] = _Hl
_CV[2] = _BASE


def _prep_body(q_ref, rp_ref, fi_ref, Wv_ref, bv_ref, Wc_ref, bc_ref,
               ExW_ref, EyH_ref, SEG_ref, cv_ref,
               val_ref, idx_ref, w_ref):
    f32 = jnp.float32
    hi = jax.lax.Precision.HIGHEST
    q = q_ref[...]
    val_ref[...] = (jnp.dot(fi_ref[...], Wv_ref[...], precision=hi)
                    + bv_ref[...]).astype(jnp.bfloat16)
    o = jnp.dot(q, Wc_ref[...], precision=hi) + bc_ref[...]
    rp = rp_ref[...]
    X = o[:, 0:128] + jnp.dot(rp, ExW_ref[...], precision=hi) - 0.5
    Y = o[:, 128:256] + jnp.dot(rp, EyH_ref[...], precision=hi) - 0.5
    a = o[:, 256:384]
    a = a - jnp.max(a, axis=-1, keepdims=True)
    e = jnp.exp(a)
    prob = e / jnp.dot(e, SEG_ref[...], precision=hi)
    WF = cv_ref[0:1, :]
    HF = cv_ref[1:2, :]
    base = cv_ref[2:3, :]
    W8 = WF * 8.0
    x0 = jnp.floor(X); fx = X - x0; x1 = x0 + 1.0
    y0 = jnp.floor(Y); fy = Y - y0; y1 = y0 + 1.0
    vx0 = ((x0 >= 0.0) & (x0 <= WF - 1.0)).astype(f32)
    vx1 = ((x1 >= 0.0) & (x1 <= WF - 1.0)).astype(f32)
    vy0 = ((y0 >= 0.0) & (y0 <= HF - 1.0)).astype(f32)
    vy1 = ((y1 >= 0.0) & (y1 <= HF - 1.0)).astype(f32)
    ix0 = jnp.clip(x0, 0.0, WF - 1.0); ix1 = jnp.clip(x1, 0.0, WF - 1.0)
    iy0 = jnp.clip(y0, 0.0, HF - 1.0); iy1 = jnp.clip(y1, 0.0, HF - 1.0)
    wx0 = 1.0 - fx; wy0 = 1.0 - fy
    for k, (ix, iy, wgt) in enumerate((
        (ix0, iy0, wx0 * wy0 * vx0 * vy0),
        (ix1, iy0, fx * wy0 * vx1 * vy0),
        (ix0, iy1, wx0 * fy * vx0 * vy1),
        (ix1, iy1, fx * fy * vx1 * vy1),
    )):
        rows = base + iy * W8 + ix * 8.0
        idx_ref[:, k * 128:(k + 1) * 128] = rows.astype(jnp.int32)
        w_ref[:, k * 128:(k + 1) * 128] = prob * wgt


def _prep(q2, rp2, fi2, Wv, bv, Wcat, bcat):
    return pl.pallas_call(
        _prep_body,
        grid=(NQB,),
        in_specs=[
            pl.BlockSpec((QBLK, D_MODEL), lambda i: (i, 0)),
            pl.BlockSpec((QBLK, 8), lambda i: (i, 0)),
            pl.BlockSpec((QBLK, D_MODEL), lambda i: (i, 0)),
            pl.BlockSpec((D_MODEL, D_MODEL), lambda i: (0, 0)),
            pl.BlockSpec((1, D_MODEL), lambda i: (0, 0)),
            pl.BlockSpec((D_MODEL, 384), lambda i: (0, 0)),
            pl.BlockSpec((1, 384), lambda i: (0, 0)),
            pl.BlockSpec((8, 128), lambda i: (0, 0)),
            pl.BlockSpec((8, 128), lambda i: (0, 0)),
            pl.BlockSpec((128, 128), lambda i: (0, 0)),
            pl.BlockSpec((8, 128), lambda i: (0, 0)),
        ],
        out_specs=[
            pl.BlockSpec((QBLK, D_MODEL), lambda i: (i, 0)),
            pl.BlockSpec((QBLK, 512), lambda i: (i, 0)),
            pl.BlockSpec((QBLK, 512), lambda i: (i, 0)),
        ],
        out_shape=[
            jax.ShapeDtypeStruct((LEN, D_MODEL), jnp.bfloat16),
            jax.ShapeDtypeStruct((LEN, 512), jnp.int32),
            jax.ShapeDtypeStruct((LEN, 512), jnp.float32),
        ],
    )(q2, rp2, fi2, Wv, bv, Wcat, bcat,
      jnp.asarray(_ExW), jnp.asarray(_EyH), jnp.asarray(_SEG),
      jnp.asarray(_CV))


def _outproj_body(s_ref, W_ref, b_ref, o_ref):
    o_ref[...] = jnp.dot(s_ref[...], W_ref[...],
                         precision=jax.lax.Precision.HIGHEST) + b_ref[...]


def _outproj(s2, Wo, bo):
    return pl.pallas_call(
        _outproj_body,
        grid=(NQB,),
        in_specs=[
            pl.BlockSpec((QBLK, D_MODEL), lambda i: (i, 0)),
            pl.BlockSpec((D_MODEL, D_MODEL), lambda i: (0, 0)),
            pl.BlockSpec((1, D_MODEL), lambda i: (0, 0)),
        ],
        out_specs=pl.BlockSpec((QBLK, D_MODEL), lambda i: (i, 0)),
        out_shape=jax.ShapeDtypeStruct((LEN, D_MODEL), jnp.float32),
    )(s2, Wo, bo)


_NC, _NS = 2, 16
_NW = _NC * _NS
_QPW = LEN // _NW


def _lane_bcast(v, i):
    dnums = lax.GatherDimensionNumbers(
        offset_dims=(), collapsed_slice_dims=(0,), start_index_map=(0,))
    return lax.gather(v, jnp.full((16, 1), i, jnp.int32), dnums, (1,),
                      mode=lax.GatherScatterMode.PROMISE_IN_BOUNDS)


_QB = 4
_NST = _QPW // _QB


def _sc_body(table, idxs, ws, out, idx_v, w_v, rows_v, acc_v, gsem, isem, osem):
    cid = lax.axis_index("c")
    sid = lax.axis_index("s")
    base = (sid * _NC + cid) * _QPW

    def iw_start(t, slot):
        pltpu.async_copy(idxs.at[pl.ds(base + t * _QB, _QB)], idx_v.at[slot],
                         isem)
        pltpu.async_copy(ws.at[pl.ds(base + t * _QB, _QB)], w_v.at[slot], isem)

    def iw_wait(slot):
        pltpu.make_async_copy(idxs.at[pl.ds(base, _QB)], idx_v.at[slot],
                              isem).wait()
        pltpu.make_async_copy(ws.at[pl.ds(base, _QB)], w_v.at[slot],
                              isem).wait()

    def g_start(iw_slot, r_slot):
        for b in range(_QB):
            for k in range(4):
                pltpu.async_copy(
                    table.at[idx_v.at[iw_slot, b, k]],
                    rows_v.at[r_slot, pl.ds((b * 4 + k) * 128, 128)], gsem)

    def g_wait(iw_slot, r_slot):
        for b in range(_QB):
            for k in range(4):
                pltpu.make_async_copy(
                    table.at[idx_v.at[iw_slot, b, k]],
                    rows_v.at[r_slot, pl.ds((b * 4 + k) * 128, 128)],
                    gsem).wait()

    pltpu.sync_copy(idxs.at[pl.ds(base, _QB)], idx_v.at[0])
    pltpu.sync_copy(ws.at[pl.ds(base, _QB)], w_v.at[0])
    g_start(0, 0)
    iw_start(1, 1)

    def per_t(t, carry):
        iw_slot = lax.rem(t, 3)
        r_slot = lax.rem(t, 2)

        @pl.when(t + 1 < _NST)
        def _():
            nxt = lax.rem(t + 1, 3)
            iw_wait(nxt)
            g_start(nxt, lax.rem(t + 1, 2))

        @pl.when(t + 2 < _NST)
        def _():
            iw_start(t + 2, lax.rem(t + 2, 3))

        g_wait(iw_slot, r_slot)

        @pl.when(t >= 2)
        def _():
            pltpu.make_async_copy(acc_v.at[r_slot],
                                  out.at[pl.ds(base, _QB)], osem).wait()

        z = jnp.zeros((16,), jnp.float32)

        def per_b(b, c2):
            rbase = b * 512
            for h in range(N_HEADS):
                def body_k(k, accs, h=h):
                    a0, a1 = accs
                    wv = w_v[iw_slot, b, k, pl.ds(h * 16, 16)]
                    rb = rbase + k * 128 + h * 16
                    pz = jnp.zeros((16,), jnp.float32)
                    p = [a0, pz, pz, pz]
                    r = [a1, pz, pz, pz]
                    for i in range(16):
                        wi = _lane_bcast(wv, i)
                        re, ro = plsc.unpack(rows_v[r_slot, rb + i, 0:32],
                                             format=plsc.PackFormat.INTERLEAVED,
                                             preferred_element_type=jnp.float32)
                        j = i & 3
                        p[j] = p[j] + re * wi
                        r[j] = r[j] + ro * wi
                    a0 = (p[0] + p[1]) + (p[2] + p[3])
                    a1 = (r[0] + r[1]) + (r[2] + r[3])
                    return (a0, a1)

                a0, a1 = lax.fori_loop(0, 4, body_k, (z, z))
                acc_v[r_slot, b, h, 0:16] = a0
                acc_v[r_slot, b, h, 16:32] = a1
            return c2

        lax.fori_loop(0, _QB, per_b, 0)
        pltpu.async_copy(acc_v.at[r_slot],
                         out.at[pl.ds(base + t * _QB, _QB)], osem)
        return carry

    lax.fori_loop(0, _NST, per_t, 0)
    pltpu.make_async_copy(acc_v.at[0], out.at[pl.ds(base, _QB)], osem).wait()
    pltpu.make_async_copy(acc_v.at[1], out.at[pl.ds(base, _QB)], osem).wait()


def _sc_sample(table, idxs, ws):
    mesh = plsc.VectorSubcoreMesh(core_axis_name="c", subcore_axis_name="s")
    f = pl.kernel(
        _sc_body,
        out_type=jax.ShapeDtypeStruct((LEN, N_HEADS, HEAD_DIM), jnp.float32),
        mesh=mesh,
        scratch_types=[
            pltpu.VMEM((3, _QB, 4, 128), jnp.int32),
            pltpu.VMEM((3, _QB, 4, 128), jnp.float32),
            pltpu.VMEM((2, _QB * 512, HEAD_DIM), jnp.bfloat16),
            pltpu.VMEM((2, _QB, N_HEADS, HEAD_DIM), jnp.float32),
            pltpu.SemaphoreType.DMA,
            pltpu.SemaphoreType.DMA,
            pltpu.SemaphoreType.DMA,
        ],
        compiler_params=pltpu.CompilerParams(use_tc_tiling_on_sc=False,
                                             needs_layout_passes=False),
    )
    return f(table, idxs, ws)


def kernel(query, reference_points, input_flatten, input_spatial_shapes,
           input_level_start_index, W_value, b_value, W_off, b_off,
           W_attn, b_attn, W_out, b_out):
    q2 = query.reshape(LEN, D_MODEL)
    rp2 = reference_points.reshape(LEN, N_LEVELS * 2)
    fi2 = input_flatten.reshape(LEN, D_MODEL)
    Wcat = jnp.concatenate([W_off[:, 0::2], W_off[:, 1::2], W_attn], axis=1)
    bcat = jnp.concatenate([b_off[0::2], b_off[1::2], b_attn]).reshape(1, 384)
    val, idx, w = _prep(q2, rp2, fi2, W_value, b_value.reshape(1, D_MODEL),
                        Wcat, bcat)
    table = val.reshape(LEN * N_HEADS, HEAD_DIM)
    sampled = _sc_sample(table, idx.reshape(LEN, 4, 128), w.reshape(LEN, 4, 128))
    Wo_p = W_out[jnp.asarray(_PERM), :]
    out = _outproj(sampled.reshape(LEN, D_MODEL), Wo_p, b_out.reshape(1, D_MODEL))
    return out.reshape(1, LEN, D_MODEL)


# speedup vs baseline: 1.0822x; 1.0822x over previous
import numpy as np
import jax
import jax.numpy as jnp
from jax import lax
from jax.experimental import pallas as pl
from jax.experimental.pallas import tpu as pltpu
from jax.experimental.pallas import tpu_sc as plsc

D_MODEL = 256
N_LEVELS = 4
N_HEADS = 8
N_POINTS = 4
HEAD_DIM = 32
_SHAPES = np.array([[128, 128], [64, 64], [32, 32], [16, 16]], dtype=np.int64)
LEN = int(_SHAPES.prod(1).sum())
_STARTS = np.concatenate([np.zeros((1,), np.int64), _SHAPES.prod(1).cumsum()[:-1]])

QBLK = 256
NQB = LEN // QBLK

_c = np.arange(N_HEADS * N_LEVELS * N_POINTS)
_l_of_c = (_c // N_POINTS) % N_LEVELS
_h_of_c = _c // (N_LEVELS * N_POINTS)
_Wl = _SHAPES[_l_of_c, 1].astype(np.float32)
_Hl = _SHAPES[_l_of_c, 0].astype(np.float32)
_BASE = (_STARTS[_l_of_c] * N_HEADS + _h_of_c).astype(np.float32)
_ExW = np.zeros((8, 128), np.float32)
_EyH = np.zeros((8, 128), np.float32)
for _cc in range(128):
    _ExW[_l_of_c[_cc] * 2 + 0, _cc] = _Wl[_cc]
    _EyH[_l_of_c[_cc] * 2 + 1, _cc] = _Hl[_cc]
_SEG = (np.arange(128)[:, None] // 16 == np.arange(128)[None, :] // 16).astype(np.float32)
_CV = np.zeros((8, 128), np.float32)
_CV[0] = _Wl
_CV[1] = _Hl
_CV[2] = _BASE


def _prep_body(q_ref, rp_ref, fi_ref, Wv_ref, bv_ref, Wc_ref, bc_ref,
               ExW_ref, EyH_ref, SEG_ref, cv_ref,
               val_ref, idx_ref, w_ref):
    f32 = jnp.float32
    hi = jax.lax.Precision.HIGHEST
    q = q_ref[...]
    val_ref[...] = (jnp.dot(fi_ref[...], Wv_ref[...])
                    + bv_ref[...]).astype(jnp.bfloat16)
    o = jnp.dot(q, Wc_ref[...], precision=hi) + bc_ref[...]
    rp = rp_ref[...]
    X = o[:, 0:128] + jnp.dot(rp, ExW_ref[...], precision=hi) - 0.5
    Y = o[:, 128:256] + jnp.dot(rp, EyH_ref[...], precision=hi) - 0.5
    a = o[:, 256:384]
    a = a - jnp.max(a, axis=-1, keepdims=True)
    e = jnp.exp(a)
    prob = e / jnp.dot(e, SEG_ref[...], precision=hi)
    WF = cv_ref[0:1, :]
    HF = cv_ref[1:2, :]
    base = cv_ref[2:3, :]
    W8 = WF * 8.0
    x0 = jnp.floor(X); fx = X - x0; x1 = x0 + 1.0
    y0 = jnp.floor(Y); fy = Y - y0; y1 = y0 + 1.0
    vx0 = ((x0 >= 0.0) & (x0 <= WF - 1.0)).astype(f32)
    vx1 = ((x1 >= 0.0) & (x1 <= WF - 1.0)).astype(f32)
    vy0 = ((y0 >= 0.0) & (y0 <= HF - 1.0)).astype(f32)
    vy1 = ((y1 >= 0.0) & (y1 <= HF - 1.0)).astype(f32)
    ix0 = jnp.clip(x0, 0.0, WF - 1.0); ix1 = jnp.clip(x1, 0.0, WF - 1.0)
    iy0 = jnp.clip(y0, 0.0, HF - 1.0); iy1 = jnp.clip(y1, 0.0, HF - 1.0)
    wx0 = 1.0 - fx; wy0 = 1.0 - fy
    for k, (ix, iy, wgt) in enumerate((
        (ix0, iy0, wx0 * wy0 * vx0 * vy0),
        (ix1, iy0, fx * wy0 * vx1 * vy0),
        (ix0, iy1, wx0 * fy * vx0 * vy1),
        (ix1, iy1, fx * fy * vx1 * vy1),
    )):
        rows = base + iy * W8 + ix * 8.0
        idx_ref[:, k * 128:(k + 1) * 128] = rows.astype(jnp.int32)
        w_ref[:, k * 128:(k + 1) * 128] = prob * wgt


def _prep(q2, rp2, fi2, Wv, bv, Wcat, bcat):
    return pl.pallas_call(
        _prep_body,
        grid=(NQB,),
        in_specs=[
            pl.BlockSpec((QBLK, D_MODEL), lambda i: (i, 0)),
            pl.BlockSpec((QBLK, 8), lambda i: (i, 0)),
            pl.BlockSpec((QBLK, D_MODEL), lambda i: (i, 0)),
            pl.BlockSpec((D_MODEL, D_MODEL), lambda i: (0, 0)),
            pl.BlockSpec((1, D_MODEL), lambda i: (0, 0)),
            pl.BlockSpec((D_MODEL, 384), lambda i: (0, 0)),
            pl.BlockSpec((1, 384), lambda i: (0, 0)),
            pl.BlockSpec((8, 128), lambda i: (0, 0)),
            pl.BlockSpec((8, 128), lambda i: (0, 0)),
            pl.BlockSpec((128, 128), lambda i: (0, 0)),
            pl.BlockSpec((8, 128), lambda i: (0, 0)),
        ],
        out_specs=[
            pl.BlockSpec((QBLK, D_MODEL), lambda i: (i, 0)),
            pl.BlockSpec((QBLK, 512), lambda i: (i, 0)),
            pl.BlockSpec((QBLK, 512), lambda i: (i, 0)),
        ],
        out_shape=[
            jax.ShapeDtypeStruct((LEN, D_MODEL), jnp.bfloat16),
            jax.ShapeDtypeStruct((LEN, 512), jnp.int32),
            jax.ShapeDtypeStruct((LEN, 512), jnp.float32),
        ],
    )(q2, rp2, fi2, Wv, bv, Wcat, bcat,
      jnp.asarray(_ExW), jnp.asarray(_EyH), jnp.asarray(_SEG),
      jnp.asarray(_CV))


def _outproj_body(s_ref, W_ref, b_ref, o_ref):
    o_ref[...] = jnp.dot(s_ref[...], W_ref[...],
                         preferred_element_type=jnp.float32) + b_ref[...]


def _outproj(s2, Wo, bo):
    return pl.pallas_call(
        _outproj_body,
        grid=(NQB,),
        in_specs=[
            pl.BlockSpec((QBLK, D_MODEL), lambda i: (i, 0)),
            pl.BlockSpec((D_MODEL, D_MODEL), lambda i: (0, 0)),
            pl.BlockSpec((1, D_MODEL), lambda i: (0, 0)),
        ],
        out_specs=pl.BlockSpec((QBLK, D_MODEL), lambda i: (i, 0)),
        out_shape=jax.ShapeDtypeStruct((LEN, D_MODEL), jnp.float32),
    )(s2, Wo, bo)


_NC, _NS = 2, 16
_NW = _NC * _NS
_QPW = LEN // _NW


def _lane_bcast(v, i):
    dnums = lax.GatherDimensionNumbers(
        offset_dims=(), collapsed_slice_dims=(0,), start_index_map=(0,))
    return lax.gather(v, jnp.full((16, 1), i, jnp.int32), dnums, (1,),
                      mode=lax.GatherScatterMode.PROMISE_IN_BOUNDS)


_QB = 4
_NST = _QPW // _QB


def _sc_body(table, idxs, ws, out, idx_v, w_v, rows_v, acc_v, gsem, isem, osem):
    cid = lax.axis_index("c")
    sid = lax.axis_index("s")
    base = (sid * _NC + cid) * _QPW

    def iw_start(t, slot):
        pltpu.async_copy(idxs.at[pl.ds(base + t * _QB, _QB)], idx_v.at[slot],
                         isem)
        pltpu.async_copy(ws.at[pl.ds(base + t * _QB, _QB)], w_v.at[slot], isem)

    def iw_wait(slot):
        pltpu.make_async_copy(idxs.at[pl.ds(base, _QB)], idx_v.at[slot],
                              isem).wait()
        pltpu.make_async_copy(ws.at[pl.ds(base, _QB)], w_v.at[slot],
                              isem).wait()

    def g_start(iw_slot, r_slot):
        for b in range(_QB):
            for k in range(4):
                pltpu.async_copy(
                    table.at[idx_v.at[iw_slot, b, k]],
                    rows_v.at[r_slot, pl.ds((b * 4 + k) * 128, 128)], gsem)

    def g_wait(iw_slot, r_slot):
        for b in range(_QB):
            for k in range(4):
                pltpu.make_async_copy(
                    table.at[idx_v.at[iw_slot, b, k]],
                    rows_v.at[r_slot, pl.ds((b * 4 + k) * 128, 128)],
                    gsem).wait()

    pltpu.sync_copy(idxs.at[pl.ds(base, _QB)], idx_v.at[0])
    pltpu.sync_copy(ws.at[pl.ds(base, _QB)], w_v.at[0])
    g_start(0, 0)
    iw_start(1, 1)

    def per_t(t, carry):
        iw_slot = lax.rem(t, 3)
        r_slot = lax.rem(t, 2)

        @pl.when(t + 1 < _NST)
        def _():
            nxt = lax.rem(t + 1, 3)
            iw_wait(nxt)
            g_start(nxt, lax.rem(t + 1, 2))

        @pl.when(t + 2 < _NST)
        def _():
            iw_start(t + 2, lax.rem(t + 2, 3))

        g_wait(iw_slot, r_slot)

        @pl.when(t >= 2)
        def _():
            pltpu.make_async_copy(acc_v.at[r_slot],
                                  out.at[pl.ds(base, _QB)], osem).wait()

        z = jnp.zeros((16,), jnp.float32)

        def per_b(b, c2):
            rbase = b * 512
            for h in range(N_HEADS):
                def body_k(k, accs, h=h):
                    a0, a1 = accs
                    wv = w_v[iw_slot, b, k, pl.ds(h * 16, 16)]
                    rb = rbase + k * 128 + h * 16
                    pz = jnp.zeros((16,), jnp.float32)
                    p = [a0, pz, pz, pz]
                    r = [a1, pz, pz, pz]
                    for i in range(16):
                        wi = _lane_bcast(wv, i)
                        re, ro = plsc.unpack(rows_v[r_slot, rb + i, 0:32],
                                             format=plsc.PackFormat.INTERLEAVED,
                                             preferred_element_type=jnp.float32)
                        j = i & 3
                        p[j] = p[j] + re * wi
                        r[j] = r[j] + ro * wi
                    a0 = (p[0] + p[1]) + (p[2] + p[3])
                    a1 = (r[0] + r[1]) + (r[2] + r[3])
                    return (a0, a1)

                a0, a1 = lax.fori_loop(0, 4, body_k, (z, z))
                acc_v[r_slot, b, pl.ds(h * 32, 32)] = plsc.pack(
                    a0, a1, format=plsc.PackFormat.INTERLEAVED)
            return c2

        lax.fori_loop(0, _QB, per_b, 0)
        pltpu.async_copy(acc_v.at[r_slot],
                         out.at[pl.ds(base + t * _QB, _QB)], osem)
        return carry

    lax.fori_loop(0, _NST, per_t, 0)
    pltpu.make_async_copy(acc_v.at[0], out.at[pl.ds(base, _QB)], osem).wait()
    pltpu.make_async_copy(acc_v.at[1], out.at[pl.ds(base, _QB)], osem).wait()


def _sc_sample(table, idxs, ws):
    mesh = plsc.VectorSubcoreMesh(core_axis_name="c", subcore_axis_name="s")
    f = pl.kernel(
        _sc_body,
        out_type=jax.ShapeDtypeStruct((LEN, D_MODEL), jnp.bfloat16),
        mesh=mesh,
        scratch_types=[
            pltpu.VMEM((3, _QB, 4, 128), jnp.int32),
            pltpu.VMEM((3, _QB, 4, 128), jnp.float32),
            pltpu.VMEM((2, _QB * 512, HEAD_DIM), jnp.bfloat16),
            pltpu.VMEM((2, _QB, D_MODEL), jnp.bfloat16),
            pltpu.SemaphoreType.DMA,
            pltpu.SemaphoreType.DMA,
            pltpu.SemaphoreType.DMA,
        ],
        compiler_params=pltpu.CompilerParams(use_tc_tiling_on_sc=False,
                                             needs_layout_passes=False),
    )
    return f(table, idxs, ws)


def kernel(query, reference_points, input_flatten, input_spatial_shapes,
           input_level_start_index, W_value, b_value, W_off, b_off,
           W_attn, b_attn, W_out, b_out):
    q2 = query.reshape(LEN, D_MODEL)
    rp2 = reference_points.reshape(LEN, N_LEVELS * 2)
    fi2 = input_flatten.reshape(LEN, D_MODEL)
    Wcat = jnp.concatenate([W_off[:, 0::2], W_off[:, 1::2], W_attn], axis=1)
    bcat = jnp.concatenate([b_off[0::2], b_off[1::2], b_attn]).reshape(1, 384)
    val, idx, w = _prep(q2, rp2, fi2, W_value, b_value.reshape(1, D_MODEL),
                        Wcat, bcat)
    table = val.reshape(LEN * N_HEADS, HEAD_DIM)
    sampled = _sc_sample(table, idx.reshape(LEN, 4, 128), w.reshape(LEN, 4, 128))
    out = _outproj(sampled, W_out.astype(jnp.bfloat16),
                   b_out.reshape(1, D_MODEL))
    return out.reshape(1, LEN, D_MODEL)


# speedup vs baseline: 1.1028x; 1.0190x over previous
import numpy as np
import jax
import jax.numpy as jnp
from jax import lax
from jax.experimental import pallas as pl
from jax.experimental.pallas import tpu as pltpu
from jax.experimental.pallas import tpu_sc as plsc

D_MODEL = 256
N_LEVELS = 4
N_HEADS = 8
N_POINTS = 4
HEAD_DIM = 32
_SHAPES = np.array([[128, 128], [64, 64], [32, 32], [16, 16]], dtype=np.int64)
LEN = int(_SHAPES.prod(1).sum())
_STARTS = np.concatenate([np.zeros((1,), np.int64), _SHAPES.prod(1).cumsum()[:-1]])

QBLK = 256
NQB = LEN // QBLK

_c = np.arange(N_HEADS * N_LEVELS * N_POINTS)
_l_of_c = (_c // N_POINTS) % N_LEVELS
_h_of_c = _c // (N_LEVELS * N_POINTS)
_Wl = _SHAPES[_l_of_c, 1].astype(np.float32)
_Hl = _SHAPES[_l_of_c, 0].astype(np.float32)
_BASE = (_STARTS[_l_of_c] * N_HEADS + _h_of_c).astype(np.float32)
_ExW = np.zeros((8, 128), np.float32)
_EyH = np.zeros((8, 128), np.float32)
for _cc in range(128):
    _ExW[_l_of_c[_cc] * 2 + 0, _cc] = _Wl[_cc]
    _EyH[_l_of_c[_cc] * 2 + 1, _cc] = _Hl[_cc]
_SEG = (np.arange(128)[:, None] // 16 == np.arange(128)[None, :] // 16).astype(np.float32)
_CV = np.zeros((8, 128), np.float32)
_CV[0] = _Wl
_CV[1] = _Hl
_CV[2] = _BASE


def _prep_body(q_ref, rp_ref, fi_ref, Wv_ref, bv_ref, Wc_ref, bc_ref,
               ExW_ref, EyH_ref, SEG_ref, cv_ref,
               val_ref, idx_ref, w_ref):
    f32 = jnp.float32
    hi = jax.lax.Precision.HIGHEST
    q = q_ref[...]
    val_ref[...] = (jnp.dot(fi_ref[...], Wv_ref[...])
                    + bv_ref[...]).astype(jnp.bfloat16)
    o = jnp.dot(q, Wc_ref[...]) + bc_ref[...]
    rp = rp_ref[...]
    X = o[:, 0:128] + jnp.dot(rp, ExW_ref[...], precision=hi) - 0.5
    Y = o[:, 128:256] + jnp.dot(rp, EyH_ref[...], precision=hi) - 0.5
    a = o[:, 256:384]
    a = a - jnp.max(a, axis=-1, keepdims=True)
    e = jnp.exp(a)
    prob = e / jnp.dot(e, SEG_ref[...], precision=hi)
    WF = cv_ref[0:1, :]
    HF = cv_ref[1:2, :]
    base = cv_ref[2:3, :]
    W8 = WF * 8.0
    x0 = jnp.floor(X); fx = X - x0; x1 = x0 + 1.0
    y0 = jnp.floor(Y); fy = Y - y0; y1 = y0 + 1.0
    vx0 = ((x0 >= 0.0) & (x0 <= WF - 1.0)).astype(f32)
    vx1 = ((x1 >= 0.0) & (x1 <= WF - 1.0)).astype(f32)
    vy0 = ((y0 >= 0.0) & (y0 <= HF - 1.0)).astype(f32)
    vy1 = ((y1 >= 0.0) & (y1 <= HF - 1.0)).astype(f32)
    ix0 = jnp.clip(x0, 0.0, WF - 1.0); ix1 = jnp.clip(x1, 0.0, WF - 1.0)
    iy0 = jnp.clip(y0, 0.0, HF - 1.0); iy1 = jnp.clip(y1, 0.0, HF - 1.0)
    wx0 = 1.0 - fx; wy0 = 1.0 - fy
    for k, (ix, iy, wgt) in enumerate((
        (ix0, iy0, wx0 * wy0 * vx0 * vy0),
        (ix1, iy0, fx * wy0 * vx1 * vy0),
        (ix0, iy1, wx0 * fy * vx0 * vy1),
        (ix1, iy1, fx * fy * vx1 * vy1),
    )):
        rows = base + iy * W8 + ix * 8.0
        idx_ref[:, k * 128:(k + 1) * 128] = rows.astype(jnp.int32)
        w_ref[:, k * 128:(k + 1) * 128] = prob * wgt


def _prep(q2, rp2, fi2, Wv, bv, Wcat, bcat):
    return pl.pallas_call(
        _prep_body,
        grid=(NQB,),
        in_specs=[
            pl.BlockSpec((QBLK, D_MODEL), lambda i: (i, 0)),
            pl.BlockSpec((QBLK, 8), lambda i: (i, 0)),
            pl.BlockSpec((QBLK, D_MODEL), lambda i: (i, 0)),
            pl.BlockSpec((D_MODEL, D_MODEL), lambda i: (0, 0)),
            pl.BlockSpec((1, D_MODEL), lambda i: (0, 0)),
            pl.BlockSpec((D_MODEL, 384), lambda i: (0, 0)),
            pl.BlockSpec((1, 384), lambda i: (0, 0)),
            pl.BlockSpec((8, 128), lambda i: (0, 0)),
            pl.BlockSpec((8, 128), lambda i: (0, 0)),
            pl.BlockSpec((128, 128), lambda i: (0, 0)),
            pl.BlockSpec((8, 128), lambda i: (0, 0)),
        ],
        out_specs=[
            pl.BlockSpec((QBLK, D_MODEL), lambda i: (i, 0)),
            pl.BlockSpec((QBLK, 512), lambda i: (i, 0)),
            pl.BlockSpec((QBLK, 512), lambda i: (i, 0)),
        ],
        out_shape=[
            jax.ShapeDtypeStruct((LEN, D_MODEL), jnp.bfloat16),
            jax.ShapeDtypeStruct((LEN, 512), jnp.int32),
            jax.ShapeDtypeStruct((LEN, 512), jnp.float32),
        ],
    )(q2, rp2, fi2, Wv, bv, Wcat, bcat,
      jnp.asarray(_ExW), jnp.asarray(_EyH), jnp.asarray(_SEG),
      jnp.asarray(_CV))


def _outproj_body(s_ref, W_ref, b_ref, o_ref):
    o_ref[...] = jnp.dot(s_ref[...], W_ref[...],
                         preferred_element_type=jnp.float32) + b_ref[...]


def _outproj(s2, Wo, bo):
    return pl.pallas_call(
        _outproj_body,
        grid=(NQB,),
        in_specs=[
            pl.BlockSpec((QBLK, D_MODEL), lambda i: (i, 0)),
            pl.BlockSpec((D_MODEL, D_MODEL), lambda i: (0, 0)),
            pl.BlockSpec((1, D_MODEL), lambda i: (0, 0)),
        ],
        out_specs=pl.BlockSpec((QBLK, D_MODEL), lambda i: (i, 0)),
        out_shape=jax.ShapeDtypeStruct((LEN, D_MODEL), jnp.float32),
    )(s2, Wo, bo)


_NC, _NS = 2, 16
_NW = _NC * _NS
_QPW = LEN // _NW


def _lane_bcast(v, i):
    dnums = lax.GatherDimensionNumbers(
        offset_dims=(), collapsed_slice_dims=(0,), start_index_map=(0,))
    return lax.gather(v, jnp.full((16, 1), i, jnp.int32), dnums, (1,),
                      mode=lax.GatherScatterMode.PROMISE_IN_BOUNDS)


_QB = 4
_NST = _QPW // _QB


def _sc_body(table, idxs, ws, out, idx_v, w_v, rows_v, acc_v, gsem, isem, osem):
    cid = lax.axis_index("c")
    sid = lax.axis_index("s")
    base = (sid * _NC + cid) * _QPW

    def iw_start(t, slot):
        pltpu.async_copy(idxs.at[pl.ds(base + t * _QB, _QB)], idx_v.at[slot],
                         isem)
        pltpu.async_copy(ws.at[pl.ds(base + t * _QB, _QB)], w_v.at[slot], isem)

    def iw_wait(slot):
        pltpu.make_async_copy(idxs.at[pl.ds(base, _QB)], idx_v.at[slot],
                              isem).wait()
        pltpu.make_async_copy(ws.at[pl.ds(base, _QB)], w_v.at[slot],
                              isem).wait()

    def g_start(iw_slot, r_slot):
        for b in range(_QB):
            for k in range(4):
                pltpu.async_copy(
                    table.at[idx_v.at[iw_slot, b, k]],
                    rows_v.at[r_slot, pl.ds((b * 4 + k) * 128, 128)], gsem)

    def g_wait(iw_slot, r_slot):
        for b in range(_QB):
            for k in range(4):
                pltpu.make_async_copy(
                    table.at[idx_v.at[iw_slot, b, k]],
                    rows_v.at[r_slot, pl.ds((b * 4 + k) * 128, 128)],
                    gsem).wait()

    pltpu.sync_copy(idxs.at[pl.ds(base, _QB)], idx_v.at[0])
    pltpu.sync_copy(ws.at[pl.ds(base, _QB)], w_v.at[0])
    g_start(0, 0)
    iw_start(1, 1)

    def per_t(t, carry):
        iw_slot = lax.rem(t, 3)
        r_slot = lax.rem(t, 2)

        @pl.when(t + 1 < _NST)
        def _():
            nxt = lax.rem(t + 1, 3)
            iw_wait(nxt)
            g_start(nxt, lax.rem(t + 1, 2))

        @pl.when(t + 2 < _NST)
        def _():
            iw_start(t + 2, lax.rem(t + 2, 3))

        g_wait(iw_slot, r_slot)

        @pl.when(t >= 2)
        def _():
            pltpu.make_async_copy(acc_v.at[r_slot],
                                  out.at[pl.ds(base, _QB)], osem).wait()

        z = jnp.zeros((16,), jnp.float32)

        def per_b(b, c2):
            rbase = b * 512
            for h in range(N_HEADS):
                def body_k(k, accs, h=h):
                    a0, a1 = accs
                    wv = w_v[iw_slot, b, k, pl.ds(h * 16, 16)]
                    rb = rbase + k * 128 + h * 16
                    pz = jnp.zeros((16,), jnp.float32)
                    p = [a0, pz, pz, pz]
                    r = [a1, pz, pz, pz]
                    for i in range(16):
                        wi = _lane_bcast(wv, i)
                        re, ro = plsc.unpack(rows_v[r_slot, rb + i, 0:32],
                                             format=plsc.PackFormat.INTERLEAVED,
                                             preferred_element_type=jnp.float32)
                        j = i & 3
                        p[j] = p[j] + re * wi
                        r[j] = r[j] + ro * wi
                    a0 = (p[0] + p[1]) + (p[2] + p[3])
                    a1 = (r[0] + r[1]) + (r[2] + r[3])
                    return (a0, a1)

                a0, a1 = lax.fori_loop(0, 4, body_k, (z, z))
                acc_v[r_slot, b, pl.ds(h * 32, 32)] = plsc.pack(
                    a0, a1, format=plsc.PackFormat.INTERLEAVED)
            return c2

        lax.fori_loop(0, _QB, per_b, 0)
        pltpu.async_copy(acc_v.at[r_slot],
                         out.at[pl.ds(base + t * _QB, _QB)], osem)
        return carry

    lax.fori_loop(0, _NST, per_t, 0)
    pltpu.make_async_copy(acc_v.at[0], out.at[pl.ds(base, _QB)], osem).wait()
    pltpu.make_async_copy(acc_v.at[1], out.at[pl.ds(base, _QB)], osem).wait()


def _sc_sample(table, idxs, ws):
    mesh = plsc.VectorSubcoreMesh(core_axis_name="c", subcore_axis_name="s")
    f = pl.kernel(
        _sc_body,
        out_type=jax.ShapeDtypeStruct((LEN, D_MODEL), jnp.bfloat16),
        mesh=mesh,
        scratch_types=[
            pltpu.VMEM((3, _QB, 4, 128), jnp.int32),
            pltpu.VMEM((3, _QB, 4, 128), jnp.float32),
            pltpu.VMEM((2, _QB * 512, HEAD_DIM), jnp.bfloat16),
            pltpu.VMEM((2, _QB, D_MODEL), jnp.bfloat16),
            pltpu.SemaphoreType.DMA,
            pltpu.SemaphoreType.DMA,
            pltpu.SemaphoreType.DMA,
        ],
        compiler_params=pltpu.CompilerParams(use_tc_tiling_on_sc=False,
                                             needs_layout_passes=False),
    )
    return f(table, idxs, ws)


def kernel(query, reference_points, input_flatten, input_spatial_shapes,
           input_level_start_index, W_value, b_value, W_off, b_off,
           W_attn, b_attn, W_out, b_out):
    q2 = query.reshape(LEN, D_MODEL)
    rp2 = reference_points.reshape(LEN, N_LEVELS * 2)
    fi2 = input_flatten.reshape(LEN, D_MODEL)
    Wcat = jnp.concatenate([W_off[:, 0::2], W_off[:, 1::2], W_attn], axis=1)
    bcat = jnp.concatenate([b_off[0::2], b_off[1::2], b_attn]).reshape(1, 384)
    val, idx, w = _prep(q2, rp2, fi2, W_value, b_value.reshape(1, D_MODEL),
                        Wcat, bcat)
    table = val.reshape(LEN * N_HEADS, HEAD_DIM)
    sampled = _sc_sample(table, idx.reshape(LEN, 4, 128), w.reshape(LEN, 4, 128))
    out = _outproj(sampled, W_out.astype(jnp.bfloat16),
                   b_out.reshape(1, D_MODEL))
    return out.reshape(1, LEN, D_MODEL)


# speedup vs baseline: 1.1892x; 1.0783x over previous
import numpy as np
import jax
import jax.numpy as jnp
from jax import lax
from jax.experimental import pallas as pl
from jax.experimental.pallas import tpu as pltpu
from jax.experimental.pallas import tpu_sc as plsc

D_MODEL = 256
N_LEVELS = 4
N_HEADS = 8
N_POINTS = 4
HEAD_DIM = 32
_SHAPES = np.array([[128, 128], [64, 64], [32, 32], [16, 16]], dtype=np.int64)
LEN = int(_SHAPES.prod(1).sum())
_STARTS = np.concatenate([np.zeros((1,), np.int64), _SHAPES.prod(1).cumsum()[:-1]])

QBLK = 256
NQB = LEN // QBLK

_c = np.arange(N_HEADS * N_LEVELS * N_POINTS)
_l_of_c = (_c // N_POINTS) % N_LEVELS
_h_of_c = _c // (N_LEVELS * N_POINTS)
_Wl = _SHAPES[_l_of_c, 1].astype(np.float32)
_Hl = _SHAPES[_l_of_c, 0].astype(np.float32)
_BASE = (_STARTS[_l_of_c] * N_HEADS + _h_of_c).astype(np.float32)
_ExW = np.zeros((8, 128), np.float32)
_EyH = np.zeros((8, 128), np.float32)
for _cc in range(128):
    _ExW[_l_of_c[_cc] * 2 + 0, _cc] = _Wl[_cc]
    _EyH[_l_of_c[_cc] * 2 + 1, _cc] = _Hl[_cc]
_SEG = (np.arange(128)[:, None] // 16 == np.arange(128)[None, :] // 16).astype(np.float32)
_CV = np.zeros((8, 128), np.float32)
_CV[0] = _Wl
_CV[1] = _Hl
_CV[2] = _BASE


def _prep_body(q_ref, rp_ref, fi_ref, Wv_ref, bv_ref, Wc_ref, bc_ref,
               ExW_ref, EyH_ref, SEG_ref, cv_ref,
               val_ref, idx_ref, w_ref):
    f32 = jnp.float32
    hi = jax.lax.Precision.HIGHEST
    q = q_ref[...]
    val_ref[...] = (jnp.dot(fi_ref[...], Wv_ref[...])
                    + bv_ref[...]).astype(jnp.bfloat16)
    o = jnp.dot(q, Wc_ref[...]) + bc_ref[...]
    rp = rp_ref[...]
    X = o[:, 0:128] + jnp.dot(rp, ExW_ref[...], precision=hi) - 0.5
    Y = o[:, 128:256] + jnp.dot(rp, EyH_ref[...], precision=hi) - 0.5
    a = o[:, 256:384]
    a = a - jnp.max(a, axis=-1, keepdims=True)
    e = jnp.exp(a)
    prob = e / jnp.dot(e, SEG_ref[...], precision=hi)
    WF = cv_ref[0:1, :]
    HF = cv_ref[1:2, :]
    base = cv_ref[2:3, :]
    W8 = WF * 8.0
    x0 = jnp.floor(X); fx = X - x0; x1 = x0 + 1.0
    y0 = jnp.floor(Y); fy = Y - y0; y1 = y0 + 1.0
    vx0 = ((x0 >= 0.0) & (x0 <= WF - 1.0)).astype(f32)
    vx1 = ((x1 >= 0.0) & (x1 <= WF - 1.0)).astype(f32)
    vy0 = ((y0 >= 0.0) & (y0 <= HF - 1.0)).astype(f32)
    vy1 = ((y1 >= 0.0) & (y1 <= HF - 1.0)).astype(f32)
    ix0 = jnp.clip(x0, 0.0, WF - 1.0); ix1 = jnp.clip(x1, 0.0, WF - 1.0)
    iy0 = jnp.clip(y0, 0.0, HF - 1.0); iy1 = jnp.clip(y1, 0.0, HF - 1.0)
    wx0 = 1.0 - fx; wy0 = 1.0 - fy
    for k, (ix, iy, wgt) in enumerate((
        (ix0, iy0, wx0 * wy0 * vx0 * vy0),
        (ix1, iy0, fx * wy0 * vx1 * vy0),
        (ix0, iy1, wx0 * fy * vx0 * vy1),
        (ix1, iy1, fx * fy * vx1 * vy1),
    )):
        rows = base + iy * W8 + ix * 8.0
        idx_ref[k, :, :] = rows.astype(jnp.int32)
        w_ref[k, :, :] = prob * wgt


def _prep(q2, rp2, fi2, Wv, bv, Wcat, bcat):
    return pl.pallas_call(
        _prep_body,
        grid=(NQB,),
        in_specs=[
            pl.BlockSpec((QBLK, D_MODEL), lambda i: (i, 0)),
            pl.BlockSpec((QBLK, 8), lambda i: (i, 0)),
            pl.BlockSpec((QBLK, D_MODEL), lambda i: (i, 0)),
            pl.BlockSpec((D_MODEL, D_MODEL), lambda i: (0, 0)),
            pl.BlockSpec((1, D_MODEL), lambda i: (0, 0)),
            pl.BlockSpec((D_MODEL, 384), lambda i: (0, 0)),
            pl.BlockSpec((1, 384), lambda i: (0, 0)),
            pl.BlockSpec((8, 128), lambda i: (0, 0)),
            pl.BlockSpec((8, 128), lambda i: (0, 0)),
            pl.BlockSpec((128, 128), lambda i: (0, 0)),
            pl.BlockSpec((8, 128), lambda i: (0, 0)),
        ],
        out_specs=[
            pl.BlockSpec((QBLK, D_MODEL), lambda i: (i, 0)),
            pl.BlockSpec((4, QBLK, 128), lambda i: (0, i, 0)),
            pl.BlockSpec((4, QBLK, 128), lambda i: (0, i, 0)),
        ],
        out_shape=[
            jax.ShapeDtypeStruct((LEN, D_MODEL), jnp.bfloat16),
            jax.ShapeDtypeStruct((4, LEN, 128), jnp.int32),
            jax.ShapeDtypeStruct((4, LEN, 128), jnp.float32),
        ],
    )(q2, rp2, fi2, Wv, bv, Wcat, bcat,
      jnp.asarray(_ExW), jnp.asarray(_EyH), jnp.asarray(_SEG),
      jnp.asarray(_CV))


def _outproj_body(s_ref, W_ref, b_ref, o_ref):
    o_ref[...] = jnp.dot(s_ref[...], W_ref[...],
                         preferred_element_type=jnp.float32) + b_ref[...]


def _outproj(s2, Wo, bo):
    return pl.pallas_call(
        _outproj_body,
        grid=(NQB,),
        in_specs=[
            pl.BlockSpec((QBLK, D_MODEL), lambda i: (i, 0)),
            pl.BlockSpec((D_MODEL, D_MODEL), lambda i: (0, 0)),
            pl.BlockSpec((1, D_MODEL), lambda i: (0, 0)),
        ],
        out_specs=pl.BlockSpec((QBLK, D_MODEL), lambda i: (i, 0)),
        out_shape=jax.ShapeDtypeStruct((LEN, D_MODEL), jnp.float32),
    )(s2, Wo, bo)


_NC, _NS = 2, 16
_NW = _NC * _NS
_QPW = LEN // _NW


def _lane_bcast(v, i):
    dnums = lax.GatherDimensionNumbers(
        offset_dims=(), collapsed_slice_dims=(0,), start_index_map=(0,))
    return lax.gather(v, jnp.full((16, 1), i, jnp.int32), dnums, (1,),
                      mode=lax.GatherScatterMode.PROMISE_IN_BOUNDS)


_QB = 4
_NST = _QPW // _QB


def _sc_body(table, idxs, ws, out, idx_v, w_v, rows_v, acc_v, gsem, isem, osem):
    cid = lax.axis_index("c")
    sid = lax.axis_index("s")
    base = (sid * _NC + cid) * _QPW

    def iw_start(t, slot):
        for k in range(4):
            pltpu.async_copy(idxs.at[k, pl.ds(base + t * _QB, _QB)],
                             idx_v.at[slot, k], isem)
            pltpu.async_copy(ws.at[k, pl.ds(base + t * _QB, _QB)],
                             w_v.at[slot, k], isem)

    def iw_wait(slot):
        for k in range(4):
            pltpu.make_async_copy(idxs.at[k, pl.ds(base, _QB)],
                                  idx_v.at[slot, k], isem).wait()
            pltpu.make_async_copy(ws.at[k, pl.ds(base, _QB)],
                                  w_v.at[slot, k], isem).wait()

    def g_start(iw_slot, r_slot):
        for b in range(_QB):
            for k in range(4):
                pltpu.async_copy(
                    table.at[idx_v.at[iw_slot, k, b]],
                    rows_v.at[r_slot, pl.ds((b * 4 + k) * 128, 128)], gsem)

    def g_wait(iw_slot, r_slot):
        for b in range(_QB):
            for k in range(4):
                pltpu.make_async_copy(
                    table.at[idx_v.at[iw_slot, k, b]],
                    rows_v.at[r_slot, pl.ds((b * 4 + k) * 128, 128)],
                    gsem).wait()

    for _k in range(4):
        pltpu.sync_copy(idxs.at[_k, pl.ds(base, _QB)], idx_v.at[0, _k])
        pltpu.sync_copy(ws.at[_k, pl.ds(base, _QB)], w_v.at[0, _k])
    g_start(0, 0)
    iw_start(1, 1)

    def per_t(t, carry):
        iw_slot = lax.rem(t, 3)
        r_slot = lax.rem(t, 2)

        @pl.when(t + 1 < _NST)
        def _():
            nxt = lax.rem(t + 1, 3)
            iw_wait(nxt)
            g_start(nxt, lax.rem(t + 1, 2))

        @pl.when(t + 2 < _NST)
        def _():
            iw_start(t + 2, lax.rem(t + 2, 3))

        g_wait(iw_slot, r_slot)

        @pl.when(t >= 2)
        def _():
            pltpu.make_async_copy(acc_v.at[r_slot],
                                  out.at[pl.ds(base, _QB)], osem).wait()

        z = jnp.zeros((16,), jnp.float32)

        def per_b(b, c2):
            rbase = b * 512
            for h in range(N_HEADS):
                def body_k(k, accs, h=h):
                    a0, a1 = accs
                    wv = w_v[iw_slot, k, b, pl.ds(h * 16, 16)]
                    rb = rbase + k * 128 + h * 16
                    pz = jnp.zeros((16,), jnp.float32)
                    p = [a0, pz, pz, pz]
                    r = [a1, pz, pz, pz]
                    for i in range(16):
                        wi = _lane_bcast(wv, i)
                        re, ro = plsc.unpack(rows_v[r_slot, rb + i, 0:32],
                                             format=plsc.PackFormat.INTERLEAVED,
                                             preferred_element_type=jnp.float32)
                        j = i & 3
                        p[j] = p[j] + re * wi
                        r[j] = r[j] + ro * wi
                    a0 = (p[0] + p[1]) + (p[2] + p[3])
                    a1 = (r[0] + r[1]) + (r[2] + r[3])
                    return (a0, a1)

                a0, a1 = lax.fori_loop(0, 4, body_k, (z, z))
                acc_v[r_slot, b, pl.ds(h * 32, 32)] = plsc.pack(
                    a0, a1, format=plsc.PackFormat.INTERLEAVED)
            return c2

        lax.fori_loop(0, _QB, per_b, 0)
        pltpu.async_copy(acc_v.at[r_slot],
                         out.at[pl.ds(base + t * _QB, _QB)], osem)
        return carry

    lax.fori_loop(0, _NST, per_t, 0)
    pltpu.make_async_copy(acc_v.at[0], out.at[pl.ds(base, _QB)], osem).wait()
    pltpu.make_async_copy(acc_v.at[1], out.at[pl.ds(base, _QB)], osem).wait()


def _sc_sample(table, idxs, ws):
    mesh = plsc.VectorSubcoreMesh(core_axis_name="c", subcore_axis_name="s")
    f = pl.kernel(
        _sc_body,
        out_type=jax.ShapeDtypeStruct((LEN, D_MODEL), jnp.bfloat16),
        mesh=mesh,
        scratch_types=[
            pltpu.VMEM((3, 4, _QB, 128), jnp.int32),
            pltpu.VMEM((3, 4, _QB, 128), jnp.float32),
            pltpu.VMEM((2, _QB * 512, HEAD_DIM), jnp.bfloat16),
            pltpu.VMEM((2, _QB, D_MODEL), jnp.bfloat16),
            pltpu.SemaphoreType.DMA,
            pltpu.SemaphoreType.DMA,
            pltpu.SemaphoreType.DMA,
        ],
        compiler_params=pltpu.CompilerParams(use_tc_tiling_on_sc=False,
                                             needs_layout_passes=False),
    )
    return f(table, idxs, ws)


def kernel(query, reference_points, input_flatten, input_spatial_shapes,
           input_level_start_index, W_value, b_value, W_off, b_off,
           W_attn, b_attn, W_out, b_out):
    q2 = query.reshape(LEN, D_MODEL)
    rp2 = reference_points.reshape(LEN, N_LEVELS * 2)
    fi2 = input_flatten.reshape(LEN, D_MODEL)
    Wcat = jnp.concatenate([W_off[:, 0::2], W_off[:, 1::2], W_attn], axis=1)
    bcat = jnp.concatenate([b_off[0::2], b_off[1::2], b_attn]).reshape(1, 384)
    val, idx, w = _prep(q2, rp2, fi2, W_value, b_value.reshape(1, D_MODEL),
                        Wcat, bcat)
    table = val.reshape(LEN * N_HEADS, HEAD_DIM)
    sampled = _sc_sample(table, idx, w)
    out = _outproj(sampled, W_out.astype(jnp.bfloat16),
                   b_out.reshape(1, D_MODEL))
    return out.reshape(1, LEN, D_MODEL)
